# Initial kernel scaffold; baseline (speedup 1.0000x reference)
import jax
import jax.numpy as jnp
from jax.experimental import pallas as pl


def kernel(x, edge_index, hyperedge_index, W_g1, b_g1, W_g2, b_g2, W_h1, b_h1, W_h2, b_h2):
    raise NotImplementedError("write your pallas kernel here")


# speedup vs baseline: 8.5726x; 8.5726x over previous
import functools

import jax
import jax.numpy as jnp
from jax import lax
from jax.experimental import pallas as pl
from jax.experimental.pallas import tpu as pltpu
from jax.experimental.pallas import tpu_sc as plsc

N = 10000
E = 320000
NE = 10000
NNZ = 320000
D = 128

NC = 2
NS = 16
NW = NC * NS
C = 80
NP = 10240
RPT = NP // NS

@functools.lru_cache(maxsize=None)
def _mesh():
    return plsc.VectorSubcoreMesh(core_axis_name="c", subcore_axis_name="s",
                                  num_cores=NC, num_subcores=NS)


@functools.lru_cache(maxsize=None)
def _make_segsum(T, E2):
    chunks = (E2 // NW) // C

    @functools.partial(
        pl.kernel,
        out_type=jax.ShapeDtypeStruct((NC, NP, D), jnp.float32),
        mesh=_mesh(),
        scratch_types=[
            pltpu.VMEM_SHARED((NP, D), jnp.float32),
            pltpu.VMEM((chunks, C), jnp.int32),
            pltpu.VMEM((chunks, C), jnp.int32),
            pltpu.VMEM((C, D), jnp.float32),
            pltpu.SemaphoreType.DMA,
        ],
    )
    def segsum(table, src3, dst3, zeros, out, acc, sidx, didx, rows, sem):
        c = lax.axis_index("c")
        s = lax.axis_index("s")
        w = c * NS + s
        pltpu.sync_copy(zeros, acc.at[pl.ds(s * RPT, RPT)])
        pltpu.sync_copy(src3.at[w], sidx)
        pltpu.sync_copy(dst3.at[w], didx)
        plsc.subcore_barrier()

        def body(j, carry):
            pltpu.async_copy(table.at[sidx.at[j]], rows, sem).wait()
            pltpu.sync_copy(rows, acc.at[didx.at[j]], add=True)
            return carry

        lax.fori_loop(0, chunks, body, 0)
        plsc.subcore_barrier()
        pltpu.sync_copy(acc.at[pl.ds(s * RPT, RPT)],
                        out.at[c, pl.ds(s * RPT, RPT)])

    return segsum


_CW = 128
_CCH = (E // NW) // C


@functools.lru_cache(maxsize=None)
def _make_counts():
    @functools.partial(
        pl.kernel,
        out_type=(
            jax.ShapeDtypeStruct((NC, NP, _CW), jnp.float32),
            jax.ShapeDtypeStruct((NC, NP, _CW), jnp.float32),
            jax.ShapeDtypeStruct((NC, NP, _CW), jnp.float32),
        ),
        mesh=_mesh(),
        scratch_types=[
            pltpu.VMEM_SHARED((NP, _CW), jnp.float32),
            pltpu.VMEM((_CCH, C), jnp.int32),
            pltpu.VMEM((C, _CW), jnp.float32),
        ],
    )
    def counts(dst3, v3, e3, zeros, ones, outd, outv, oute,
               acc, idx, ones_v):
        c = lax.axis_index("c")
        s = lax.axis_index("s")
        w = c * NS + s
        sl = pl.ds(s * RPT, RPT)
        pltpu.sync_copy(ones, ones_v)
        for idx3, out in ((dst3, outd), (v3, outv), (e3, oute)):
            pltpu.sync_copy(zeros, acc.at[sl])
            pltpu.sync_copy(idx3.at[w], idx)
            plsc.subcore_barrier()

            def body(j, carry):
                pltpu.sync_copy(ones_v, acc.at[idx.at[j]], add=True)
                return carry

            lax.fori_loop(0, _CCH, body, 0)
            plsc.subcore_barrier()
            pltpu.sync_copy(acc.at[sl], out.at[c, sl])
            plsc.subcore_barrier()

    return counts


_B = 1000
_GRID = N // _B


def _row_spec(nrow=_B, ncol=D):
    return pl.BlockSpec((nrow, ncol), lambda i: (i, 0))


def _part_spec(ncol=D):
    return pl.BlockSpec((NC, _B, ncol), lambda i: (0, i, 0))


def _full_spec(shape):
    nd = len(shape)
    return pl.BlockSpec(shape, lambda i: (0,) * nd)


def _tc1_body(x, wg1, wh1, bh1, degp, vcp, ecp,
              hg1p, hh1, dinv_b, vinv_b, einv_b):
    deg = degp[0, :, :1] + degp[1, :, :1] + 1.0
    dinv = lax.rsqrt(deg)
    dinv_b[...] = jnp.broadcast_to(dinv, (_B, D))
    vinv_b[...] = jnp.broadcast_to(
        1.0 / jnp.maximum(vcp[0, :, :1] + vcp[1, :, :1], 1.0), (_B, D))
    einv_b[...] = jnp.broadcast_to(
        1.0 / jnp.maximum(ecp[0, :, :1] + ecp[1, :, :1], 1.0), (_B, D))
    hg1p[...] = jnp.dot(x[...], wg1[...],
                        preferred_element_type=jnp.float32) * dinv_b[...]
    hh1[...] = jnp.dot(x[...], wh1[...],
                       preferred_element_type=jnp.float32) + bh1[...]


def _tc2_body(sg1p, hg1p, bg1, dinv_b, se1p, einv_b, wg2, hg2p, ef1):
    x1 = jnp.maximum(
        dinv_b[...] * (sg1p[0] + sg1p[1] + hg1p[...]) + bg1[...], 0.0)
    hg2p[...] = jnp.dot(x1, wg2[...],
                        preferred_element_type=jnp.float32) * dinv_b[...]
    ef1[...] = (se1p[0] + se1p[1]) * einv_b[...]


def _tc3_body(sg2p, hg2p, bg2, dinv_b, sv1p, vinv_b, wh2, bh2, x2, hh2):
    x2[...] = dinv_b[...] * (sg2p[0] + sg2p[1] + hg2p[...]) + bg2[...]
    x3 = jnp.maximum((sv1p[0] + sv1p[1]) * vinv_b[...], 0.0)
    hh2[...] = jnp.dot(x3, wh2[...],
                       preferred_element_type=jnp.float32) + bh2[...]


def _tc4_body(se2p, einv_b, ef2):
    ef2[...] = (se2p[0] + se2p[1]) * einv_b[...]


def _tc5_body(sv2p, vinv_b, x2, out):
    out[...] = 0.5 * x2[...] + 0.5 * (sv2p[0] + sv2p[1]) * vinv_b[...]


def _row_out(k=1):
    o = [jax.ShapeDtypeStruct((N, D), jnp.float32) for _ in range(k)]
    return o[0] if k == 1 else tuple(o)


_tc1 = pl.pallas_call(
    _tc1_body,
    grid=(_GRID,),
    in_specs=[_row_spec(), _full_spec((D, D)), _full_spec((D, D)),
              _full_spec((D,)), _part_spec(_CW), _part_spec(_CW),
              _part_spec(_CW)],
    out_specs=[_row_spec()] * 5,
    out_shape=_row_out(5),
)

_tc2 = pl.pallas_call(
    _tc2_body,
    grid=(_GRID,),
    in_specs=[_part_spec(), _row_spec(), _full_spec((D,)), _row_spec(),
              _part_spec(), _row_spec(), _full_spec((D, D))],
    out_specs=[_row_spec()] * 2,
    out_shape=_row_out(2),
)

_tc3 = pl.pallas_call(
    _tc3_body,
    grid=(_GRID,),
    in_specs=[_part_spec(), _row_spec(), _full_spec((D,)), _row_spec(),
              _part_spec(), _row_spec(), _full_spec((D, D)),
              _full_spec((D,))],
    out_specs=[_row_spec()] * 2,
    out_shape=_row_out(2),
)

_tc4 = pl.pallas_call(
    _tc4_body,
    grid=(_GRID,),
    in_specs=[_part_spec(), _row_spec()],
    out_specs=_row_spec(),
    out_shape=_row_out(),
)

_tc5 = pl.pallas_call(
    _tc5_body,
    grid=(_GRID,),
    in_specs=[_part_spec(), _row_spec(), _row_spec()],
    out_specs=_row_spec(),
    out_shape=_row_out(),
)


def kernel(x, edge_index, hyperedge_index,
           W_g1, b_g1, W_g2, b_g2, W_h1, b_h1, W_h2, b_h2):
    src2 = edge_index[0].reshape(NW, E // NW // C, C)
    dst2 = edge_index[1].reshape(NW, E // NW // C, C)
    v2 = hyperedge_index[0].reshape(NW, NNZ // NW // C, C)
    e2 = hyperedge_index[1].reshape(NW, NNZ // NW // C, C)

    zeros_d = jnp.zeros((RPT, D), jnp.float32)
    zeros_c = jnp.zeros((RPT, _CW), jnp.float32)
    ones_c = jnp.ones((C, _CW), jnp.float32)

    segsum_nodes = _make_segsum(N, E)
    segsum_v2e = _make_segsum(N, NNZ)
    segsum_e2v = _make_segsum(NE, NNZ)

    degp, vcp, ecp = _make_counts()(dst2, v2, e2, zeros_c, ones_c)
    hg1p, hh1, dinv_b, vinv_b, einv_b = _tc1(
        x, W_g1, W_h1, b_h1, degp, vcp, ecp)

    sg1p = segsum_nodes(hg1p, src2, dst2, zeros_d)
    se1p = segsum_v2e(hh1, v2, e2, zeros_d)
    hg2p, ef1 = _tc2(sg1p, hg1p, b_g1, dinv_b, se1p, einv_b, W_g2)

    sg2p = segsum_nodes(hg2p, src2, dst2, zeros_d)
    sv1p = segsum_e2v(ef1, e2, v2, zeros_d)
    x2, hh2 = _tc3(sg2p, hg2p, b_g2, dinv_b, sv1p, vinv_b, W_h2, b_h2)

    se2p = segsum_v2e(hh2, v2, e2, zeros_d)
    ef2 = _tc4(se2p, einv_b)

    sv2p = segsum_e2v(ef2, e2, v2, zeros_d)
    return _tc5(sv2p, vinv_b, x2)


# speedup vs baseline: 13.2084x; 1.5408x over previous
import functools

import jax
import jax.numpy as jnp
from jax import lax
from jax.experimental import pallas as pl
from jax.experimental.pallas import tpu as pltpu
from jax.experimental.pallas import tpu_sc as plsc

N = 10000
E = 320000
NE = 10000
NNZ = 320000
D = 128

NC = 2
NS = 16
NW = NC * NS
C = 100
NP = 10240
RPT = NP // NS

@functools.lru_cache(maxsize=None)
def _mesh():
    return plsc.VectorSubcoreMesh(core_axis_name="c", subcore_axis_name="s",
                                  num_cores=NC, num_subcores=NS)


_PH = 2


@functools.lru_cache(maxsize=None)
def _make_segsum(T, E2):
    chunks = (E2 // NW) // C
    cph = chunks // _PH

    @functools.partial(
        pl.kernel,
        out_type=jax.ShapeDtypeStruct((NC, NP, D), jnp.float32),
        mesh=_mesh(),
        scratch_types=[
            pltpu.VMEM_SHARED((NP, D), jnp.float32),
            pltpu.VMEM((cph, C), jnp.int32),
            pltpu.VMEM((cph, C), jnp.int32),
            pltpu.VMEM((C, D), jnp.float32),
            pltpu.VMEM((C, D), jnp.float32),
            pltpu.SemaphoreType.DMA,
            pltpu.SemaphoreType.DMA,
        ],
    )
    def segsum(table, src4, dst4, zeros, out,
               acc, sidx, didx, rows0, rows1, sem0, sem1):
        c = lax.axis_index("c")
        s = lax.axis_index("s")
        w = c * NS + s
        pltpu.sync_copy(zeros, acc.at[pl.ds(s * RPT, RPT)])
        plsc.subcore_barrier()

        rows = (rows0, rows1)
        sems = (sem0, sem1)
        for phase in range(_PH):
            pltpu.sync_copy(src4.at[w, phase], sidx)
            pltpu.sync_copy(dst4.at[w, phase], didx)
            pltpu.async_copy(table.at[sidx.at[0]], rows0, sem0)
            pltpu.async_copy(table.at[sidx.at[1]], rows1, sem1)

            def body(i, carry):
                for b in range(2):
                    j = 2 * i + b
                    pltpu.make_async_copy(table.at[sidx.at[j]],
                                          rows[b], sems[b]).wait()
                    pltpu.sync_copy(rows[b], acc.at[didx.at[j]], add=True)

                    @pl.when(j + 2 < cph)
                    def _():
                        pltpu.async_copy(table.at[sidx.at[j + 2]],
                                         rows[b], sems[b])
                return carry

            lax.fori_loop(0, cph // 2, body, 0)
        plsc.subcore_barrier()
        pltpu.sync_copy(acc.at[pl.ds(s * RPT, RPT)],
                        out.at[c, pl.ds(s * RPT, RPT)])

    return segsum


_CW = 128
_CCH = (E // NW) // C


@functools.lru_cache(maxsize=None)
def _make_counts():
    @functools.partial(
        pl.kernel,
        out_type=(
            jax.ShapeDtypeStruct((NC, NP, _CW), jnp.float32),
            jax.ShapeDtypeStruct((NC, NP, _CW), jnp.float32),
            jax.ShapeDtypeStruct((NC, NP, _CW), jnp.float32),
        ),
        mesh=_mesh(),
        scratch_types=[
            pltpu.VMEM_SHARED((NP, _CW), jnp.float32),
            pltpu.VMEM((_CCH // _PH, C), jnp.int32),
            pltpu.VMEM((C, _CW), jnp.float32),
        ],
    )
    def counts(dst4, v4, e4, zeros, ones, outd, outv, oute,
               acc, idx, ones_v):
        c = lax.axis_index("c")
        s = lax.axis_index("s")
        w = c * NS + s
        sl = pl.ds(s * RPT, RPT)
        cph = _CCH // _PH
        pltpu.sync_copy(ones, ones_v)
        for idx4, out in ((dst4, outd), (v4, outv), (e4, oute)):
            pltpu.sync_copy(zeros, acc.at[sl])
            plsc.subcore_barrier()
            for phase in range(_PH):
                pltpu.sync_copy(idx4.at[w, phase], idx)

                def body(j, carry):
                    pltpu.sync_copy(ones_v, acc.at[idx.at[j]], add=True)
                    return carry

                lax.fori_loop(0, cph, body, 0)
            plsc.subcore_barrier()
            pltpu.sync_copy(acc.at[sl], out.at[c, sl])
            plsc.subcore_barrier()

    return counts


_B = 1000
_GRID = N // _B


def _row_spec(nrow=_B, ncol=D):
    return pl.BlockSpec((nrow, ncol), lambda i: (i, 0))


def _part_spec(ncol=D):
    return pl.BlockSpec((NC, _B, ncol), lambda i: (0, i, 0))


def _full_spec(shape):
    nd = len(shape)
    return pl.BlockSpec(shape, lambda i: (0,) * nd)


def _tc1_body(x, wg1, wh1, bh1, degp, vcp, ecp,
              hg1p, hh1, dinv_b, vinv_b, einv_b):
    deg = degp[0, :, :1] + degp[1, :, :1] + 1.0
    dinv = lax.rsqrt(deg)
    dinv_b[...] = jnp.broadcast_to(dinv, (_B, D))
    vinv_b[...] = jnp.broadcast_to(
        1.0 / jnp.maximum(vcp[0, :, :1] + vcp[1, :, :1], 1.0), (_B, D))
    einv_b[...] = jnp.broadcast_to(
        1.0 / jnp.maximum(ecp[0, :, :1] + ecp[1, :, :1], 1.0), (_B, D))
    hg1p[...] = jnp.dot(x[...], wg1[...],
                        preferred_element_type=jnp.float32) * dinv_b[...]
    hh1[...] = jnp.dot(x[...], wh1[...],
                       preferred_element_type=jnp.float32) + bh1[...]


def _tc2_body(sg1p, hg1p, bg1, dinv_b, se1p, einv_b, wg2, hg2p, ef1):
    x1 = jnp.maximum(
        dinv_b[...] * (sg1p[0] + sg1p[1] + hg1p[...]) + bg1[...], 0.0)
    hg2p[...] = jnp.dot(x1, wg2[...],
                        preferred_element_type=jnp.float32) * dinv_b[...]
    ef1[...] = (se1p[0] + se1p[1]) * einv_b[...]


def _tc3_body(sg2p, hg2p, bg2, dinv_b, sv1p, vinv_b, wh2, bh2, x2, hh2):
    x2[...] = dinv_b[...] * (sg2p[0] + sg2p[1] + hg2p[...]) + bg2[...]
    x3 = jnp.maximum((sv1p[0] + sv1p[1]) * vinv_b[...], 0.0)
    hh2[...] = jnp.dot(x3, wh2[...],
                       preferred_element_type=jnp.float32) + bh2[...]


def _tc4_body(se2p, einv_b, ef2):
    ef2[...] = (se2p[0] + se2p[1]) * einv_b[...]


def _tc5_body(sv2p, vinv_b, x2, out):
    out[...] = 0.5 * x2[...] + 0.5 * (sv2p[0] + sv2p[1]) * vinv_b[...]


def _row_out(k=1):
    o = [jax.ShapeDtypeStruct((N, D), jnp.float32) for _ in range(k)]
    return o[0] if k == 1 else tuple(o)


_tc1 = pl.pallas_call(
    _tc1_body,
    grid=(_GRID,),
    in_specs=[_row_spec(), _full_spec((D, D)), _full_spec((D, D)),
              _full_spec((D,)), _part_spec(_CW), _part_spec(_CW),
              _part_spec(_CW)],
    out_specs=[_row_spec()] * 5,
    out_shape=_row_out(5),
)

_tc2 = pl.pallas_call(
    _tc2_body,
    grid=(_GRID,),
    in_specs=[_part_spec(), _row_spec(), _full_spec((D,)), _row_spec(),
              _part_spec(), _row_spec(), _full_spec((D, D))],
    out_specs=[_row_spec()] * 2,
    out_shape=_row_out(2),
)

_tc3 = pl.pallas_call(
    _tc3_body,
    grid=(_GRID,),
    in_specs=[_part_spec(), _row_spec(), _full_spec((D,)), _row_spec(),
              _part_spec(), _row_spec(), _full_spec((D, D)),
              _full_spec((D,))],
    out_specs=[_row_spec()] * 2,
    out_shape=_row_out(2),
)

_tc4 = pl.pallas_call(
    _tc4_body,
    grid=(_GRID,),
    in_specs=[_part_spec(), _row_spec()],
    out_specs=_row_spec(),
    out_shape=_row_out(),
)

_tc5 = pl.pallas_call(
    _tc5_body,
    grid=(_GRID,),
    in_specs=[_part_spec(), _row_spec(), _row_spec()],
    out_specs=_row_spec(),
    out_shape=_row_out(),
)


def kernel(x, edge_index, hyperedge_index,
           W_g1, b_g1, W_g2, b_g2, W_h1, b_h1, W_h2, b_h2):
    src2 = edge_index[0].reshape(NW, _PH, E // NW // C // _PH, C)
    dst2 = edge_index[1].reshape(NW, _PH, E // NW // C // _PH, C)
    v2 = hyperedge_index[0].reshape(NW, _PH, NNZ // NW // C // _PH, C)
    e2 = hyperedge_index[1].reshape(NW, _PH, NNZ // NW // C // _PH, C)

    zeros_d = jnp.zeros((RPT, D), jnp.float32)
    zeros_c = jnp.zeros((RPT, _CW), jnp.float32)
    ones_c = jnp.ones((C, _CW), jnp.float32)

    segsum_nodes = _make_segsum(N, E)
    segsum_v2e = _make_segsum(N, NNZ)
    segsum_e2v = _make_segsum(NE, NNZ)

    degp, vcp, ecp = _make_counts()(dst2, v2, e2, zeros_c, ones_c)
    hg1p, hh1, dinv_b, vinv_b, einv_b = _tc1(
        x, W_g1, W_h1, b_h1, degp, vcp, ecp)

    sg1p = segsum_nodes(hg1p, src2, dst2, zeros_d)
    se1p = segsum_v2e(hh1, v2, e2, zeros_d)
    hg2p, ef1 = _tc2(sg1p, hg1p, b_g1, dinv_b, se1p, einv_b, W_g2)

    sg2p = segsum_nodes(hg2p, src2, dst2, zeros_d)
    sv1p = segsum_e2v(ef1, e2, v2, zeros_d)
    x2, hh2 = _tc3(sg2p, hg2p, b_g2, dinv_b, sv1p, vinv_b, W_h2, b_h2)

    se2p = segsum_v2e(hh2, v2, e2, zeros_d)
    ef2 = _tc4(se2p, einv_b)

    sv2p = segsum_e2v(ef2, e2, v2, zeros_d)
    return _tc5(sv2p, vinv_b, x2)


# speedup vs baseline: 13.4289x; 1.0167x over previous
import functools

import jax
import jax.numpy as jnp
from jax import lax
from jax.experimental import pallas as pl
from jax.experimental.pallas import tpu as pltpu
from jax.experimental.pallas import tpu_sc as plsc

N = 10000
E = 320000
NE = 10000
NNZ = 320000
D = 128

NC = 2
NS = 16
NW = NC * NS
C = 125
NP = 10240
RPT = NP // NS

@functools.lru_cache(maxsize=None)
def _mesh():
    return plsc.VectorSubcoreMesh(core_axis_name="c", subcore_axis_name="s",
                                  num_cores=NC, num_subcores=NS)


_PH = 2


@functools.lru_cache(maxsize=None)
def _make_segsum(T, E2):
    chunks = (E2 // NW) // C
    cph = chunks // _PH

    @functools.partial(
        pl.kernel,
        out_type=jax.ShapeDtypeStruct((NC, NP, D), jnp.float32),
        mesh=_mesh(),
        scratch_types=[
            pltpu.VMEM_SHARED((NP, D), jnp.float32),
            pltpu.VMEM((cph, C), jnp.int32),
            pltpu.VMEM((cph, C), jnp.int32),
            pltpu.VMEM((C, D), jnp.float32),
            pltpu.VMEM((C, D), jnp.float32),
            pltpu.SemaphoreType.DMA,
            pltpu.SemaphoreType.DMA,
        ],
    )
    def segsum(table, src4, dst4, zeros, out,
               acc, sidx, didx, rows0, rows1, sem0, sem1):
        c = lax.axis_index("c")
        s = lax.axis_index("s")
        w = c * NS + s
        pltpu.sync_copy(zeros, acc.at[pl.ds(s * RPT, RPT)])
        plsc.subcore_barrier()

        rows = (rows0, rows1)
        sems = (sem0, sem1)
        for phase in range(_PH):
            pltpu.sync_copy(src4.at[w, phase], sidx)
            pltpu.sync_copy(dst4.at[w, phase], didx)
            pltpu.async_copy(table.at[sidx.at[0]], rows0, sem0)
            pltpu.async_copy(table.at[sidx.at[1]], rows1, sem1)

            def body(i, carry):
                for b in range(2):
                    j = 2 * i + b
                    pltpu.make_async_copy(table.at[sidx.at[j]],
                                          rows[b], sems[b]).wait()
                    pltpu.sync_copy(rows[b], acc.at[didx.at[j]], add=True)

                    @pl.when(j + 2 < cph)
                    def _():
                        pltpu.async_copy(table.at[sidx.at[j + 2]],
                                         rows[b], sems[b])
                return carry

            lax.fori_loop(0, cph // 2, body, 0)
        plsc.subcore_barrier()
        pltpu.sync_copy(acc.at[pl.ds(s * RPT, RPT)],
                        out.at[c, pl.ds(s * RPT, RPT)])

    return segsum


_CW = 128
_CCH = (E // NW) // C


@functools.lru_cache(maxsize=None)
def _make_counts():
    @functools.partial(
        pl.kernel,
        out_type=(
            jax.ShapeDtypeStruct((NC, NP, _CW), jnp.float32),
            jax.ShapeDtypeStruct((NC, NP, _CW), jnp.float32),
            jax.ShapeDtypeStruct((NC, NP, _CW), jnp.float32),
        ),
        mesh=_mesh(),
        scratch_types=[
            pltpu.VMEM_SHARED((NP, _CW), jnp.float32),
            pltpu.VMEM((_CCH // _PH, C), jnp.int32),
            pltpu.VMEM((C, _CW), jnp.float32),
        ],
    )
    def counts(dst4, v4, e4, zeros, ones, outd, outv, oute,
               acc, idx, ones_v):
        c = lax.axis_index("c")
        s = lax.axis_index("s")
        w = c * NS + s
        sl = pl.ds(s * RPT, RPT)
        cph = _CCH // _PH
        pltpu.sync_copy(ones, ones_v)
        for idx4, out in ((dst4, outd), (v4, outv), (e4, oute)):
            pltpu.sync_copy(zeros, acc.at[sl])
            plsc.subcore_barrier()
            for phase in range(_PH):
                pltpu.sync_copy(idx4.at[w, phase], idx)

                def body(j, carry):
                    pltpu.sync_copy(ones_v, acc.at[idx.at[j]], add=True)
                    return carry

                lax.fori_loop(0, cph, body, 0)
            plsc.subcore_barrier()
            pltpu.sync_copy(acc.at[sl], out.at[c, sl])
            plsc.subcore_barrier()

    return counts


_B = 1000
_GRID = N // _B


def _row_spec(nrow=_B, ncol=D):
    return pl.BlockSpec((nrow, ncol), lambda i: (i, 0))


def _part_spec(ncol=D):
    return pl.BlockSpec((NC, _B, ncol), lambda i: (0, i, 0))


def _full_spec(shape):
    nd = len(shape)
    return pl.BlockSpec(shape, lambda i: (0,) * nd)


def _tc1_body(x, wg1, wh1, bh1, degp, vcp, ecp,
              hg1p, hh1, dinv_b, vinv_b, einv_b):
    deg = degp[0, :, :1] + degp[1, :, :1] + 1.0
    dinv = lax.rsqrt(deg)
    dinv_b[...] = jnp.broadcast_to(dinv, (_B, D))
    vinv_b[...] = jnp.broadcast_to(
        1.0 / jnp.maximum(vcp[0, :, :1] + vcp[1, :, :1], 1.0), (_B, D))
    einv_b[...] = jnp.broadcast_to(
        1.0 / jnp.maximum(ecp[0, :, :1] + ecp[1, :, :1], 1.0), (_B, D))
    hg1p[...] = jnp.dot(x[...], wg1[...],
                        preferred_element_type=jnp.float32) * dinv_b[...]
    hh1[...] = jnp.dot(x[...], wh1[...],
                       preferred_element_type=jnp.float32) + bh1[...]


def _tc2_body(sg1p, hg1p, bg1, dinv_b, se1p, einv_b, wg2, hg2p, ef1):
    x1 = jnp.maximum(
        dinv_b[...] * (sg1p[0] + sg1p[1] + hg1p[...]) + bg1[...], 0.0)
    hg2p[...] = jnp.dot(x1, wg2[...],
                        preferred_element_type=jnp.float32) * dinv_b[...]
    ef1[...] = (se1p[0] + se1p[1]) * einv_b[...]


def _tc3_body(sg2p, hg2p, bg2, dinv_b, sv1p, vinv_b, wh2, bh2, x2, hh2):
    x2[...] = dinv_b[...] * (sg2p[0] + sg2p[1] + hg2p[...]) + bg2[...]
    x3 = jnp.maximum((sv1p[0] + sv1p[1]) * vinv_b[...], 0.0)
    hh2[...] = jnp.dot(x3, wh2[...],
                       preferred_element_type=jnp.float32) + bh2[...]


def _tc4_body(se2p, einv_b, ef2):
    ef2[...] = (se2p[0] + se2p[1]) * einv_b[...]


def _tc5_body(sv2p, vinv_b, x2, out):
    out[...] = 0.5 * x2[...] + 0.5 * (sv2p[0] + sv2p[1]) * vinv_b[...]


def _row_out(k=1):
    o = [jax.ShapeDtypeStruct((N, D), jnp.float32) for _ in range(k)]
    return o[0] if k == 1 else tuple(o)


_tc1 = pl.pallas_call(
    _tc1_body,
    grid=(_GRID,),
    in_specs=[_row_spec(), _full_spec((D, D)), _full_spec((D, D)),
              _full_spec((D,)), _part_spec(_CW), _part_spec(_CW),
              _part_spec(_CW)],
    out_specs=[_row_spec()] * 5,
    out_shape=_row_out(5),
)

_tc2 = pl.pallas_call(
    _tc2_body,
    grid=(_GRID,),
    in_specs=[_part_spec(), _row_spec(), _full_spec((D,)), _row_spec(),
              _part_spec(), _row_spec(), _full_spec((D, D))],
    out_specs=[_row_spec()] * 2,
    out_shape=_row_out(2),
)

_tc3 = pl.pallas_call(
    _tc3_body,
    grid=(_GRID,),
    in_specs=[_part_spec(), _row_spec(), _full_spec((D,)), _row_spec(),
              _part_spec(), _row_spec(), _full_spec((D, D)),
              _full_spec((D,))],
    out_specs=[_row_spec()] * 2,
    out_shape=_row_out(2),
)

_tc4 = pl.pallas_call(
    _tc4_body,
    grid=(_GRID,),
    in_specs=[_part_spec(), _row_spec()],
    out_specs=_row_spec(),
    out_shape=_row_out(),
)

_tc5 = pl.pallas_call(
    _tc5_body,
    grid=(_GRID,),
    in_specs=[_part_spec(), _row_spec(), _row_spec()],
    out_specs=_row_spec(),
    out_shape=_row_out(),
)


def kernel(x, edge_index, hyperedge_index,
           W_g1, b_g1, W_g2, b_g2, W_h1, b_h1, W_h2, b_h2):
    src2 = edge_index[0].reshape(NW, _PH, E // NW // C // _PH, C)
    dst2 = edge_index[1].reshape(NW, _PH, E // NW // C // _PH, C)
    v2 = hyperedge_index[0].reshape(NW, _PH, NNZ // NW // C // _PH, C)
    e2 = hyperedge_index[1].reshape(NW, _PH, NNZ // NW // C // _PH, C)

    zeros_d = jnp.zeros((RPT, D), jnp.float32)
    zeros_c = jnp.zeros((RPT, _CW), jnp.float32)
    ones_c = jnp.ones((C, _CW), jnp.float32)

    segsum_nodes = _make_segsum(N, E)
    segsum_v2e = _make_segsum(N, NNZ)
    segsum_e2v = _make_segsum(NE, NNZ)

    degp, vcp, ecp = _make_counts()(dst2, v2, e2, zeros_c, ones_c)
    hg1p, hh1, dinv_b, vinv_b, einv_b = _tc1(
        x, W_g1, W_h1, b_h1, degp, vcp, ecp)

    sg1p = segsum_nodes(hg1p, src2, dst2, zeros_d)
    se1p = segsum_v2e(hh1, v2, e2, zeros_d)
    hg2p, ef1 = _tc2(sg1p, hg1p, b_g1, dinv_b, se1p, einv_b, W_g2)

    sg2p = segsum_nodes(hg2p, src2, dst2, zeros_d)
    sv1p = segsum_e2v(ef1, e2, v2, zeros_d)
    x2, hh2 = _tc3(sg2p, hg2p, b_g2, dinv_b, sv1p, vinv_b, W_h2, b_h2)

    se2p = segsum_v2e(hh2, v2, e2, zeros_d)
    ef2 = _tc4(se2p, einv_b)

    sv2p = segsum_e2v(ef2, e2, v2, zeros_d)
    return _tc5(sv2p, vinv_b, x2)
